# baseline (speedup 1.0000x reference)
import functools

import numpy as np
import jax
import jax.numpy as jnp
from jax import lax
from jax.experimental import pallas as pl
from jax.experimental.pallas import tpu as pltpu
from jax.experimental.pallas import tpu_sc as plsc

B = 64
N = 8192
NN = N - 1
EMB = 32
TAGS = 32
PADROW = 63
G = 4
NG = B // G
LW = G * EMB

_NC = 2
_NS = 16
_NW = _NC * _NS
_CHUNK = 128


def _bitrev(q, bits):
    r = 0
    for _ in range(bits):
        r = (r << 1) | (q & 1)
        q >>= 1
    return r


@functools.cache
def _perm():
    g = np.zeros(N, np.int64)
    for d in range(6):
        for q in range(2 ** d):
            g[2 ** d - 1 + q] = 2 ** d - 1 + _bitrev(q, d)
    g[PADROW] = NN
    for d in range(6, 13):
        for q in range(2 ** d):
            g[2 ** d + q] = 2 ** d - 1 + _bitrev(q, d)
    ginv = np.zeros(N, np.int64)
    ginv[g] = np.arange(N)
    return g, ginv


@functools.cache
def _top_mats():
    g, ginv = _perm()
    A = np.zeros((4, 128, 256), np.float32)
    for p in range(128):
        i = int(g[p])
        if i == NN:
            nbrs = [0, 0, 0, 0]
        else:
            par = (i - 1) // 2 if i > 0 else NN
            c0, c1 = 2 * i + 1, 2 * i + 2
            if c1 >= NN:
                c0 = c1 = NN
            nbrs = [i, par, c0, c1]
        for t, nb in enumerate(nbrs):
            q = int(ginv[nb])
            assert q < 256
            A[t, p, q] = 1.0
    return A


def _sc_gather_body(rows_pw, table_hbm, idx_hbm, out_hbm, idx_v, buf_v, sem):
    wid = lax.axis_index("s") * _NC + lax.axis_index("c")
    row0 = wid * rows_pw
    pltpu.sync_copy(idx_hbm.at[pl.ds(row0, rows_pw)], idx_v)
    pltpu.async_copy(table_hbm.at[idx_v.at[0]], buf_v.at[0], sem.at[0])

    def chunk(c, carry):
        cur = lax.rem(c, 2)
        nxt = lax.rem(c + 1, 2)

        @pl.when(c + 1 < rows_pw)
        def _():
            pltpu.async_copy(table_hbm.at[idx_v.at[c + 1]], buf_v.at[nxt],
                             sem.at[nxt])

        pltpu.make_async_copy(table_hbm.at[idx_v.at[c]], buf_v.at[cur],
                              sem.at[cur]).wait()
        pltpu.sync_copy(buf_v.at[cur],
                        out_hbm.at[pl.ds((row0 + c) * _CHUNK, _CHUNK)])
        return carry

    lax.fori_loop(0, rows_pw, chunk, 0)


def _sc_gather(table, idx, tc_tiling=False):
    R = idx.shape[0]
    assert R % (_CHUNK * _NW) == 0
    rows_pw = R // _CHUNK // _NW
    width = table.shape[1]
    kfn = pl.kernel(
        functools.partial(_sc_gather_body, rows_pw),
        out_type=jax.ShapeDtypeStruct((R, width), table.dtype),
        mesh=plsc.VectorSubcoreMesh(core_axis_name="c", subcore_axis_name="s"),
        scratch_types=[
            pltpu.VMEM((rows_pw, _CHUNK), jnp.int32),
            pltpu.VMEM((2, _CHUNK, width), table.dtype),
            pltpu.SemaphoreType.DMA((2,)),
        ],
        compiler_params=pltpu.CompilerParams(use_tc_tiling_on_sc=tc_tiling),
    )
    return kfn(table, idx.reshape(-1, _CHUNK))


def _sc_gather4_body(chunks_pw, table_hbm, idx_hbm, out_hbm, idx_v, buf_v,
                     sem):
    wid = lax.axis_index("s") * _NC + lax.axis_index("c")
    c0 = wid * chunks_pw
    pltpu.sync_copy(idx_hbm.at[pl.ds(c0 * G, chunks_pw * G)], idx_v)

    def fire(t, slot):
        for j in range(G):
            pltpu.async_copy(table_hbm.at[idx_v.at[t * G + j]],
                             buf_v.at[slot, j], sem.at[slot])

    fire(0, 0)

    def chunk(t, carry):
        cur = lax.rem(t, 2)
        nxt = lax.rem(t + 1, 2)

        @pl.when(t + 1 < chunks_pw)
        def _():
            fire(t + 1, nxt)

        for j in range(G):
            pltpu.make_async_copy(table_hbm.at[idx_v.at[t * G + j]],
                                  buf_v.at[cur, j], sem.at[cur]).wait()
        row = (c0 + t) * _CHUNK
        for j in range(G):
            pltpu.sync_copy(buf_v.at[cur, j],
                            out_hbm.at[pl.ds(row, _CHUNK),
                                       pl.ds(EMB * j, EMB)])
        return carry

    lax.fori_loop(0, chunks_pw, chunk, 0)


def _sc_gather_packed(table, idx4):
    chunks = idx4.shape[0] // G
    assert chunks % _NW == 0
    chunks_pw = chunks // _NW
    width = table.shape[1]
    kfn = pl.kernel(
        functools.partial(_sc_gather4_body, chunks_pw),
        out_type=jax.ShapeDtypeStruct((chunks * _CHUNK, G * width),
                                      table.dtype),
        mesh=plsc.VectorSubcoreMesh(core_axis_name="c", subcore_axis_name="s"),
        scratch_types=[
            pltpu.VMEM((chunks_pw * G, _CHUNK), jnp.int32),
            pltpu.VMEM((2, G, _CHUNK, width), table.dtype),
            pltpu.SemaphoreType.DMA((2,)),
        ],
        compiler_params=pltpu.CompilerParams(use_tc_tiling_on_sc=False),
    )
    return kfn(table, idx4)


def _layer(h, WBD, bias, As, Ap, Ac0, Ac1):
    Z = jnp.dot(h.astype(jnp.bfloat16), WBD,
                preferred_element_type=jnp.float32)
    S = Z[:, 0:128]
    P = Z[:, 128:256]
    C0 = Z[:, 256:384]
    C1 = Z[:, 384:512]
    top = (jnp.dot(As, S[0:256], preferred_element_type=jnp.float32)
           + jnp.dot(Ap, P[0:256], preferred_element_type=jnp.float32)
           + jnp.dot(Ac0, C0[0:256], preferred_element_type=jnp.float32)
           + jnp.dot(Ac1, C1[0:256], preferred_element_type=jnp.float32))
    e = (lax.broadcasted_iota(jnp.int32, (1, 64), 1) == PADROW)
    zpad = jnp.dot(e.astype(jnp.float32), Z[0:64],
                   preferred_element_type=jnp.float32)
    PT = jnp.concatenate(
        [P[64:128], P[64:128], P[128:256], P[128:256], P[256:512],
         P[256:512], P[512:1024], P[512:1024], P[1024:2048], P[1024:2048],
         P[2048:4096], P[2048:4096]], axis=0)
    CT0 = jnp.concatenate(
        [C0[256:384], C0[512:768], C0[1024:1536], C0[2048:3072],
         C0[4096:6144], jnp.broadcast_to(zpad[:, 256:384], (4096, LW))],
        axis=0)
    CT1 = jnp.concatenate(
        [C1[384:512], C1[768:1024], C1[1536:2048], C1[3072:4096],
         C1[6144:8192], jnp.broadcast_to(zpad[:, 384:512], (4096, LW))],
        axis=0)
    tail = S[128:N] + PT + CT0 + CT1
    return jnp.concatenate([top, tail], axis=0) + bias


def _leaky(v):
    return jnp.where(v >= 0, v, 0.01 * v)


def _tc_body(h_ref, w0_ref, b0_ref, w1_ref, b1_ref, we_ref, be_ref,
             as_ref, ap_ref, ac0_ref, ac1_ref, mg_ref, out_ref):
    h = h_ref[0]
    A = (as_ref[...], ap_ref[...], ac0_ref[...], ac1_ref[...])
    a0 = _leaky(_layer(h, w0_ref[...], b0_ref[...], *A))
    a1 = _leaky(_layer(a0, w1_ref[...], b1_ref[...], *A))
    ye = _layer(a1, we_ref[...], be_ref[...], *A)
    m = jnp.max(ye, axis=1, keepdims=True)
    s = ye - m
    denom = jnp.dot(jnp.exp(s).astype(jnp.bfloat16), mg_ref[...],
                    preferred_element_type=jnp.float32)
    out_ref[0] = s - jnp.log(denom)


def _tc_call(h4, WBD0, b0, WBD1, b1, WBDe, be, interpret=False):
    ng = h4.shape[0]
    Atop = jnp.asarray(_top_mats())
    mgrp = jnp.asarray(
        (np.arange(LW)[:, None] // EMB == np.arange(LW)[None, :] // EMB)
        .astype(np.float32), jnp.bfloat16)
    wspec = pl.BlockSpec((LW, 4 * LW), lambda b: (0, 0))
    bspec = pl.BlockSpec((1, LW), lambda b: (0, 0))
    aspec = pl.BlockSpec((128, 256), lambda b: (0, 0))
    return pl.pallas_call(
        _tc_body,
        grid=(ng,),
        in_specs=[
            pl.BlockSpec((1, N, LW), lambda b: (b, 0, 0)),
            wspec, bspec, wspec, bspec, wspec, bspec,
            aspec, aspec, aspec, aspec,
            pl.BlockSpec((LW, LW), lambda b: (0, 0)),
        ],
        out_specs=pl.BlockSpec((1, N, LW), lambda b: (b, 0, 0)),
        out_shape=jax.ShapeDtypeStruct((ng, N, LW), jnp.float32),
        interpret=interpret,
    )(h4, WBD0, b0, WBD1, b1, WBDe, be,
      Atop[0], Atop[1], Atop[2], Atop[3], mgrp)


def _wbd(W, out):
    Wt = W.reshape(4, EMB, out)
    eye = jnp.eye(G, dtype=W.dtype)
    M = jnp.concatenate([jnp.kron(eye, Wt[t]) for t in range(4)], axis=1)
    return M.astype(jnp.bfloat16)


def _bias4(b):
    return jnp.tile(jnp.asarray(b, jnp.float32), G).reshape(1, LW)


@jax.jit
def kernel(x, indices, emb, W0, b0, W1, b1, We, be):
    del indices
    g, ginv = _perm()
    xPT = _sc_gather(x.T, jnp.asarray(g, jnp.int32))
    idx4 = (xPT.reshape(N // _CHUNK, _CHUNK, NG, G)
            .transpose(2, 0, 3, 1).reshape(NG, -1, _CHUNK))
    W = (_wbd(W0, EMB), _bias4(b0), _wbd(W1, EMB), _bias4(b1),
         _wbd(We, TAGS), _bias4(be))
    S = 4
    NGk = NG // S
    J = jnp.asarray((np.arange(NGk)[:, None] * N + ginv[None, :])
                    .reshape(-1), jnp.int32)
    outs = []
    for k in range(S):
        h = _sc_gather_packed(
            emb, idx4[k * NGk:(k + 1) * NGk].reshape(-1, _CHUNK))
        y = _tc_call(h.reshape(NGk, N, LW), *W)
        y_nat = _sc_gather(y.reshape(NGk * N, LW), J, tc_tiling=True)
        outs.append(y_nat.reshape(NGk, N, G, TAGS)
                    .transpose(0, 2, 3, 1).reshape(NGk * G, TAGS, N))
    return jnp.concatenate(outs, axis=0)


# speedup vs baseline: 1.3019x; 1.3019x over previous
import functools

import numpy as np
import jax
import jax.numpy as jnp
from jax import lax
from jax.experimental import pallas as pl
from jax.experimental.pallas import tpu as pltpu
from jax.experimental.pallas import tpu_sc as plsc

B = 64
N = 8192
NN = N - 1
EMB = 32
TAGS = 32
PADROW = 63
G = 4
NG = B // G
LW = G * EMB

_NC = 2
_NS = 16
_NW = _NC * _NS
_CHUNK = 128


def _bitrev(q, bits):
    r = 0
    for _ in range(bits):
        r = (r << 1) | (q & 1)
        q >>= 1
    return r


@functools.cache
def _perm():
    g = np.zeros(N, np.int64)
    for d in range(6):
        for q in range(2 ** d):
            g[2 ** d - 1 + q] = 2 ** d - 1 + _bitrev(q, d)
    g[PADROW] = NN
    for d in range(6, 13):
        for q in range(2 ** d):
            g[2 ** d + q] = 2 ** d - 1 + _bitrev(q, d)
    ginv = np.zeros(N, np.int64)
    ginv[g] = np.arange(N)
    return g, ginv


@functools.cache
def _top_mats():
    g, ginv = _perm()
    A = np.zeros((4, 128, 256), np.float32)
    for p in range(128):
        i = int(g[p])
        if i == NN:
            nbrs = [0, 0, 0, 0]
        else:
            par = (i - 1) // 2 if i > 0 else NN
            c0, c1 = 2 * i + 1, 2 * i + 2
            if c1 >= NN:
                c0 = c1 = NN
            nbrs = [i, par, c0, c1]
        for t, nb in enumerate(nbrs):
            q = int(ginv[nb])
            assert q < 256
            A[t, p, q] = 1.0
    return A


def _sc_gather_body(rows_pw, table_hbm, idx_hbm, out_hbm, idx_v, buf_v, sem):
    wid = lax.axis_index("s") * _NC + lax.axis_index("c")
    row0 = wid * rows_pw
    pltpu.sync_copy(idx_hbm.at[pl.ds(row0, rows_pw)], idx_v)
    pltpu.async_copy(table_hbm.at[idx_v.at[0]], buf_v.at[0], sem.at[0])

    def chunk(c, carry):
        cur = lax.rem(c, 2)
        nxt = lax.rem(c + 1, 2)

        @pl.when(c + 1 < rows_pw)
        def _():
            pltpu.async_copy(table_hbm.at[idx_v.at[c + 1]], buf_v.at[nxt],
                             sem.at[nxt])

        pltpu.make_async_copy(table_hbm.at[idx_v.at[c]], buf_v.at[cur],
                              sem.at[cur]).wait()
        pltpu.sync_copy(buf_v.at[cur],
                        out_hbm.at[pl.ds((row0 + c) * _CHUNK, _CHUNK)])
        return carry

    lax.fori_loop(0, rows_pw, chunk, 0)


def _sc_gather(table, idx, tc_tiling=False):
    R = idx.shape[0]
    assert R % (_CHUNK * _NW) == 0
    rows_pw = R // _CHUNK // _NW
    width = table.shape[1]
    kfn = pl.kernel(
        functools.partial(_sc_gather_body, rows_pw),
        out_type=jax.ShapeDtypeStruct((R, width), table.dtype),
        mesh=plsc.VectorSubcoreMesh(core_axis_name="c", subcore_axis_name="s"),
        scratch_types=[
            pltpu.VMEM((rows_pw, _CHUNK), jnp.int32),
            pltpu.VMEM((2, _CHUNK, width), table.dtype),
            pltpu.SemaphoreType.DMA((2,)),
        ],
        compiler_params=pltpu.CompilerParams(use_tc_tiling_on_sc=tc_tiling),
    )
    return kfn(table, idx.reshape(-1, _CHUNK))


def _sc_gather4_body(chunks_pw, table_hbm, idx_hbm, out_hbm, idx_v, buf_v,
                     sem):
    wid = lax.axis_index("s") * _NC + lax.axis_index("c")
    c0 = wid * chunks_pw
    pltpu.sync_copy(idx_hbm.at[pl.ds(c0 * G, chunks_pw * G)], idx_v)

    def fire(t, slot):
        for j in range(G):
            pltpu.async_copy(table_hbm.at[idx_v.at[t * G + j]],
                             buf_v.at[slot, j], sem.at[slot])

    fire(0, 0)

    def chunk(t, carry):
        cur = lax.rem(t, 2)
        nxt = lax.rem(t + 1, 2)

        @pl.when(t + 1 < chunks_pw)
        def _():
            fire(t + 1, nxt)

        for j in range(G):
            pltpu.make_async_copy(table_hbm.at[idx_v.at[t * G + j]],
                                  buf_v.at[cur, j], sem.at[cur]).wait()
        row = (c0 + t) * _CHUNK
        for j in range(G):
            pltpu.sync_copy(buf_v.at[cur, j],
                            out_hbm.at[pl.ds(row, _CHUNK),
                                       pl.ds(EMB * j, EMB)])
        return carry

    lax.fori_loop(0, chunks_pw, chunk, 0)


def _sc_gather_packed(table, idx4):
    chunks = idx4.shape[0] // G
    assert chunks % _NW == 0
    chunks_pw = chunks // _NW
    width = table.shape[1]
    kfn = pl.kernel(
        functools.partial(_sc_gather4_body, chunks_pw),
        out_type=jax.ShapeDtypeStruct((chunks * _CHUNK, G * width),
                                      table.dtype),
        mesh=plsc.VectorSubcoreMesh(core_axis_name="c", subcore_axis_name="s"),
        scratch_types=[
            pltpu.VMEM((chunks_pw * G, _CHUNK), jnp.int32),
            pltpu.VMEM((2, G, _CHUNK, width), table.dtype),
            pltpu.SemaphoreType.DMA((2,)),
        ],
        compiler_params=pltpu.CompilerParams(use_tc_tiling_on_sc=False),
    )
    return kfn(table, idx4)


def _layer(h, WBD, bias, As, Ap, Ac0, Ac1):
    Z = jnp.dot(h.astype(jnp.bfloat16), WBD,
                preferred_element_type=jnp.float32)
    S = Z[:, 0:128]
    P = Z[:, 128:256]
    C0 = Z[:, 256:384]
    C1 = Z[:, 384:512]
    top = (jnp.dot(As, S[0:256], preferred_element_type=jnp.float32)
           + jnp.dot(Ap, P[0:256], preferred_element_type=jnp.float32)
           + jnp.dot(Ac0, C0[0:256], preferred_element_type=jnp.float32)
           + jnp.dot(Ac1, C1[0:256], preferred_element_type=jnp.float32))
    e = (lax.broadcasted_iota(jnp.int32, (1, 64), 1) == PADROW)
    zpad = jnp.dot(e.astype(jnp.float32), Z[0:64],
                   preferred_element_type=jnp.float32)
    PT = jnp.concatenate(
        [P[64:128], P[64:128], P[128:256], P[128:256], P[256:512],
         P[256:512], P[512:1024], P[512:1024], P[1024:2048], P[1024:2048],
         P[2048:4096], P[2048:4096]], axis=0)
    CT0 = jnp.concatenate(
        [C0[256:384], C0[512:768], C0[1024:1536], C0[2048:3072],
         C0[4096:6144], jnp.broadcast_to(zpad[:, 256:384], (4096, LW))],
        axis=0)
    CT1 = jnp.concatenate(
        [C1[384:512], C1[768:1024], C1[1536:2048], C1[3072:4096],
         C1[6144:8192], jnp.broadcast_to(zpad[:, 384:512], (4096, LW))],
        axis=0)
    tail = S[128:N] + PT + CT0 + CT1
    return jnp.concatenate([top, tail], axis=0) + bias


def _leaky(v):
    return jnp.where(v >= 0, v, 0.01 * v)


def _tc_body(h_ref, w0_ref, b0_ref, w1_ref, b1_ref, we_ref, be_ref,
             as_ref, ap_ref, ac0_ref, ac1_ref, mg_ref, out_ref):
    h = h_ref[0]
    A = (as_ref[...], ap_ref[...], ac0_ref[...], ac1_ref[...])
    a0 = _leaky(_layer(h, w0_ref[...], b0_ref[...], *A))
    a1 = _leaky(_layer(a0, w1_ref[...], b1_ref[...], *A))
    ye = _layer(a1, we_ref[...], be_ref[...], *A)
    m = jnp.max(ye, axis=1, keepdims=True)
    s = ye - m
    denom = jnp.dot(jnp.exp(s).astype(jnp.bfloat16), mg_ref[...],
                    preferred_element_type=jnp.float32)
    out_ref[0] = s - jnp.log(denom)


def _tc_call(h4, WBD0, b0, WBD1, b1, WBDe, be, interpret=False):
    Atop = jnp.asarray(_top_mats())
    mgrp = jnp.asarray(
        (np.arange(LW)[:, None] // EMB == np.arange(LW)[None, :] // EMB)
        .astype(np.float32), jnp.bfloat16)
    wspec = pl.BlockSpec((LW, 4 * LW), lambda b: (0, 0))
    bspec = pl.BlockSpec((1, LW), lambda b: (0, 0))
    aspec = pl.BlockSpec((128, 256), lambda b: (0, 0))
    return pl.pallas_call(
        _tc_body,
        grid=(NG,),
        in_specs=[
            pl.BlockSpec((1, N, LW), lambda b: (b, 0, 0)),
            wspec, bspec, wspec, bspec, wspec, bspec,
            aspec, aspec, aspec, aspec,
            pl.BlockSpec((LW, LW), lambda b: (0, 0)),
        ],
        out_specs=pl.BlockSpec((1, N, LW), lambda b: (b, 0, 0)),
        out_shape=jax.ShapeDtypeStruct((NG, N, LW), jnp.float32),
        interpret=interpret,
    )(h4, WBD0, b0, WBD1, b1, WBDe, be,
      Atop[0], Atop[1], Atop[2], Atop[3], mgrp)


def _tc_xpose_body(y_ref, out_ref):
    out_ref[0] = y_ref[0].T


def _tc_xpose(y3):
    return pl.pallas_call(
        _tc_xpose_body,
        grid=(NG,),
        in_specs=[pl.BlockSpec((1, N, LW), lambda b: (b, 0, 0))],
        out_specs=pl.BlockSpec((1, LW, N), lambda b: (b, 0, 0)),
        out_shape=jax.ShapeDtypeStruct((NG, LW, N), jnp.float32),
    )(y3)


def _wbd(W, out):
    Wt = W.reshape(4, EMB, out)
    eye = jnp.eye(G, dtype=W.dtype)
    M = jnp.concatenate([jnp.kron(eye, Wt[t]) for t in range(4)], axis=1)
    return M.astype(jnp.bfloat16)


def _bias4(b):
    return jnp.tile(jnp.asarray(b, jnp.float32), G).reshape(1, LW)


@jax.jit
def kernel(x, indices, emb, W0, b0, W1, b1, We, be):
    del indices
    g, ginv = _perm()
    xPT = _sc_gather(x.T, jnp.asarray(g, jnp.int32))
    idx4 = (xPT.reshape(N // _CHUNK, _CHUNK, NG, G)
            .transpose(2, 0, 3, 1).reshape(-1, _CHUNK))
    h = _sc_gather_packed(emb, idx4)
    y = _tc_call(
        h.reshape(NG, N, LW),
        _wbd(W0, EMB), _bias4(b0),
        _wbd(W1, EMB), _bias4(b1),
        _wbd(We, TAGS), _bias4(be),
    )
    J = (np.arange(NG)[:, None] * N + ginv[None, :]).reshape(-1)
    y_nat = _sc_gather(y.reshape(NG * N, LW), jnp.asarray(J, jnp.int32),
                       tc_tiling=True)
    yt = _tc_xpose(y_nat.reshape(NG, N, LW))
    return yt.reshape(B, TAGS, N)
